# baseline (speedup 1.0000x reference)
import functools

import jax
import jax.numpy as jnp
from jax import lax
from jax.experimental import pallas as pl
from jax.experimental.pallas import tpu as pltpu
from jax.experimental.pallas import tpu_sc as plsc


@functools.lru_cache(maxsize=None)
def _make_sc_gather(batch, seq, vocab, dim):
    info = plsc.get_sparse_core_info()
    num_cores = info.num_cores
    num_workers = info.num_cores * info.num_subcores
    tiles_b, tiles_s = batch // 8, seq // 128
    assert tiles_b * tiles_s == num_workers
    mesh = plsc.VectorSubcoreMesh(core_axis_name="c", subcore_axis_name="s")

    @functools.partial(
        pl.kernel,
        mesh=mesh,
        out_type=jax.ShapeDtypeStruct((dim, batch, seq), jnp.float32),
        scratch_types=[
            pltpu.VMEM((8, 128), jnp.int32),
            pltpu.VMEM((8, 128), jnp.float32),
            pltpu.SemaphoreType.DMA,
        ],
        compiler_params=pltpu.CompilerParams(use_tc_tiling_on_sc=False),
    )
    def sc_gather(ids_hbm, tab_t_hbm, out_t_hbm, idx_v, gath_v, gsem):
        wid = lax.axis_index("s") * num_cores + lax.axis_index("c")
        b0 = 8 * (wid // tiles_s)
        s0 = 128 * (wid % tiles_s)
        pltpu.sync_copy(ids_hbm.at[pl.ds(b0, 8), pl.ds(s0, 128)], idx_v)

        def body(j, carry):
            handles = [
                pltpu.async_copy(
                    tab_t_hbm.at[j].at[idx_v.at[r]], gath_v.at[r], gsem)
                for r in range(8)
            ]
            for h in handles:
                h.wait()
            pltpu.sync_copy(
                gath_v, out_t_hbm.at[j, pl.ds(b0, 8), pl.ds(s0, 128)])
            return carry

        lax.fori_loop(0, dim, body, 0)

    return sc_gather


def kernel(input_ids, embedding_table):
    batch, seq = input_ids.shape
    vocab, dim = embedding_table.shape
    ids = input_ids.astype(jnp.int32)
    out_t = _make_sc_gather(batch, seq, vocab, dim)(ids, embedding_table.T)
    output = out_t.transpose(1, 2, 0)
    return (output, embedding_table)


# speedup vs baseline: 6.0888x; 6.0888x over previous
import functools

import jax
import jax.numpy as jnp
from jax import lax
from jax.experimental import pallas as pl
from jax.experimental.pallas import tpu as pltpu
from jax.experimental.pallas import tpu_sc as plsc


@functools.lru_cache(maxsize=None)
def _make_sc_gather(batch, seq, vocab, dim):
    info = plsc.get_sparse_core_info()
    num_cores = info.num_cores
    num_workers = info.num_cores * info.num_subcores
    tiles_s = seq // 128
    assert (batch // 8) * tiles_s == num_workers
    mesh = plsc.VectorSubcoreMesh(core_axis_name="c", subcore_axis_name="s")
    lanes = 2 * dim

    @functools.partial(
        pl.kernel,
        mesh=mesh,
        out_type=jax.ShapeDtypeStruct((batch, seq, lanes), jnp.float32),
        scratch_types=[
            pltpu.VMEM((8, 128), jnp.int32),
            pltpu.VMEM((8, 128), jnp.int32),
            pltpu.VMEM((8, 128), jnp.int32),
            pltpu.VMEM((128, lanes), jnp.float32),
            pltpu.VMEM((128, lanes), jnp.float32),
            pltpu.SemaphoreType.DMA,
        ],
        compiler_params=pltpu.CompilerParams(use_tc_tiling_on_sc=True,
                                             needs_layout_passes=False),
    )
    def sc_gather(ids_hbm, tab2_hbm, out_hbm, ids_v, row_v, half_v, gath_v,
                  ext_v, gsem):
        wid = lax.axis_index("s") * num_cores + lax.axis_index("c")
        b0 = 8 * (wid // tiles_s)
        s0 = 128 * (wid % tiles_s)
        pltpu.sync_copy(ids_hbm.at[pl.ds(b0, 8), pl.ds(s0, 128)], ids_v)
        for r in range(8):
            for m in range(8):
                sl = (r, pl.ds(16 * m, 16))
                v = ids_v[sl]
                row_v[sl] = v >> 1
                half_v[sl] = (v & 1) << 6
        for r in range(8):
            pltpu.async_copy(tab2_hbm.at[row_v.at[r]], gath_v, gsem).wait()
            for g in range(8):
                rows16 = 16 * g + lax.iota(jnp.int32, 16)
                halves = half_v[r, pl.ds(16 * g, 16)]

                def body(j, carry, rows16=rows16, halves=halves):
                    vals = plsc.load_gather(gath_v, [rows16, halves + j])
                    plsc.store_scatter(
                        ext_v, [rows16, jnp.full((16,), j, jnp.int32)], vals)
                    return carry

                lax.fori_loop(0, dim, body, 0)
            pltpu.sync_copy(ext_v,
                            out_hbm.at[b0 + r].at[pl.ds(s0, 128)])
        return None

    return sc_gather


def kernel(input_ids, embedding_table):
    batch, seq = input_ids.shape
    vocab, dim = embedding_table.shape
    ids = input_ids.astype(jnp.int32)
    tab2 = embedding_table.reshape(vocab // 2, 2 * dim)
    out128 = _make_sc_gather(batch, seq, vocab, dim)(ids, tab2)
    output = out128[:, :, :dim]
    return (output, embedding_table)
